# Initial kernel scaffold; baseline (speedup 1.0000x reference)
import jax
import jax.numpy as jnp
from jax.experimental import pallas as pl


def kernel(x, params, edge_index, batch):
    raise NotImplementedError("write your pallas kernel here")


# speedup vs baseline: 1.0067x; 1.0067x over previous
import jax
import jax.numpy as jnp
from jax.experimental import pallas as pl

_HID = 64
_NL = 8
_RATIO = 0.8
_NG = 64
_BLK = 2000


def _conv_kernel(mean_ref, h_ref, wrel_ref, brel_ref, wroot_ref, mask_ref, o_ref):
    out = (
        jnp.dot(mean_ref[...], wrel_ref[...].T, preferred_element_type=jnp.float32)
        + brel_ref[...]
        + jnp.dot(h_ref[...], wroot_ref[...].T, preferred_element_type=jnp.float32)
    )
    o_ref[...] = jnp.maximum(out, 0.0) * mask_ref[...]


def _conv_dense(mean, h, wrel, brel, wroot, maskf):
    n = h.shape[0]
    return pl.pallas_call(
        _conv_kernel,
        grid=(n // _BLK,),
        in_specs=[
            pl.BlockSpec((_BLK, _HID), lambda i: (i, 0)),
            pl.BlockSpec((_BLK, _HID), lambda i: (i, 0)),
            pl.BlockSpec((_HID, _HID), lambda i: (0, 0)),
            pl.BlockSpec((1, _HID), lambda i: (0, 0)),
            pl.BlockSpec((_HID, _HID), lambda i: (0, 0)),
            pl.BlockSpec((_BLK, 1), lambda i: (i, 0)),
        ],
        out_specs=pl.BlockSpec((_BLK, _HID), lambda i: (i, 0)),
        out_shape=jax.ShapeDtypeStruct((n, _HID), jnp.float32),
    )(mean, h, wrel, brel.reshape(1, _HID), wroot, maskf)


def _score_kernel(h_ref, p_ref, o_ref):
    o_ref[...] = jnp.sum(h_ref[...] * p_ref[...], axis=1, keepdims=True)


def _score(h, p):
    n = h.shape[0]
    return pl.pallas_call(
        _score_kernel,
        grid=(n // _BLK,),
        in_specs=[
            pl.BlockSpec((_BLK, _HID), lambda i: (i, 0)),
            pl.BlockSpec((1, _HID), lambda i: (0, 0)),
        ],
        out_specs=pl.BlockSpec((_BLK, 1), lambda i: (i, 0)),
        out_shape=jax.ShapeDtypeStruct((n, 1), jnp.float32),
    )(h, p.reshape(1, _HID))


def _head_kernel(xc_ref, w1_ref, b1_ref, w2_ref, b2_ref, o_ref):
    z = jnp.maximum(
        jnp.dot(xc_ref[...], w1_ref[...].T, preferred_element_type=jnp.float32)
        + b1_ref[...],
        0.0,
    )
    out = jnp.dot(z, w2_ref[...].T, preferred_element_type=jnp.float32) + b2_ref[...]
    m = jnp.max(out, axis=1, keepdims=True)
    lse = m + jnp.log(jnp.sum(jnp.exp(out - m), axis=1, keepdims=True))
    o_ref[...] = out - lse


def _head(xc, w1, b1, w2, b2):
    return pl.pallas_call(
        _head_kernel,
        out_shape=jax.ShapeDtypeStruct((xc.shape[0], w2.shape[0]), jnp.float32),
    )(xc, w1, b1.reshape(1, -1), w2, b2.reshape(1, -1))


def _agg_mean(h, src, dst, edge_mask, n):
    em = edge_mask.astype(jnp.float32)
    msgs = h[src] * em[:, None]
    agg = jax.ops.segment_sum(msgs, dst, num_segments=n)
    deg = jax.ops.segment_sum(em, dst, num_segments=n)
    return agg / jnp.clip(deg, 1.0)[:, None]


def _gmp(h, batch, maskf):
    s = jax.ops.segment_sum(h, batch, num_segments=_NG)
    c = jax.ops.segment_sum(maskf[:, 0], batch, num_segments=_NG)
    return s / jnp.clip(c, 1.0)[:, None]


def _topk(h, p, batch, node_mask, edge_mask, src, dst, starts):
    n = h.shape[0]
    score = _score(h, p)[:, 0] / (jnp.linalg.norm(p) + 1e-16)
    score_m = jnp.where(node_mask, score, -jnp.inf)
    o1 = jnp.argsort(-score_m)
    order = o1[jnp.argsort(batch[o1])]
    pos = jnp.zeros(n, dtype=jnp.int32).at[order].set(jnp.arange(n, dtype=jnp.int32))
    rank = pos - starts[batch]
    active = jax.ops.segment_sum(node_mask.astype(jnp.float32), batch, num_segments=_NG)
    k = jnp.ceil(_RATIO * active).astype(jnp.int32)
    keep = (rank < k[batch]) & node_mask
    x_new = h * jnp.tanh(score)[:, None] * keep[:, None].astype(h.dtype)
    return x_new, keep, edge_mask & keep[src] & keep[dst]


def kernel(x, params, edge_index, batch):
    src, dst = edge_index[0], edge_index[1]
    n = x.shape[0]
    cnt = jnp.bincount(batch, length=_NG)
    starts = (jnp.cumsum(cnt) - cnt).astype(jnp.int32)
    node_mask = jnp.ones(n, dtype=bool)
    edge_mask = jnp.ones(src.shape[0], dtype=bool)
    maskf = node_mask.astype(jnp.float32)[:, None]

    wrel1 = jnp.zeros((_HID, _HID), jnp.float32).at[:, 0].set(params["Wrel1"][:, 0])
    wroot1 = jnp.zeros((_HID, _HID), jnp.float32).at[:, 0].set(params["Wroot1"][:, 0])
    mean0 = jnp.pad(_agg_mean(x, src, dst, edge_mask, n), ((0, 0), (0, _HID - 1)))
    xp = jnp.pad(x, ((0, 0), (0, _HID - 1)))
    h = _conv_dense(mean0, xp, wrel1, params["brel1"], wroot1, maskf)

    xs = [_gmp(h, batch, maskf)]
    for i in range(_NL - 1):
        mean = _agg_mean(h, src, dst, edge_mask, n)
        h = _conv_dense(mean, h, params["Wrel"][i], params["brel"][i], params["Wroot"][i], maskf)
        xs.append(_gmp(h, batch, maskf))
        if i % 2 == 0 and i < _NL - 2:
            h, node_mask, edge_mask = _topk(
                h, params["pool_p"][i // 2], batch, node_mask, edge_mask, src, dst, starts
            )
            maskf = node_mask.astype(jnp.float32)[:, None]

    xc = jnp.concatenate(xs, axis=1)
    return _head(xc, params["W_lin1"], params["b_lin1"], params["W_lin2"], params["b_lin2"])


# speedup vs baseline: 1.2519x; 1.2436x over previous
import jax
import jax.numpy as jnp
from jax.experimental import pallas as pl

_HID = 64
_NL = 8
_RATIO = 0.8
_NG = 64
_BLK = 2000


def _conv_kernel(mean_ref, h_ref, wrel_ref, brel_ref, wroot_ref, mask_ref, o_ref):
    out = (
        jnp.dot(mean_ref[...], wrel_ref[...].T, preferred_element_type=jnp.float32)
        + brel_ref[...]
        + jnp.dot(h_ref[...], wroot_ref[...].T, preferred_element_type=jnp.float32)
    )
    o_ref[...] = jnp.maximum(out, 0.0) * mask_ref[...]


def _conv_dense(mean, h, wrel, brel, wroot, maskf):
    n = h.shape[0]
    return pl.pallas_call(
        _conv_kernel,
        grid=(n // _BLK,),
        in_specs=[
            pl.BlockSpec((_BLK, _HID), lambda i: (i, 0)),
            pl.BlockSpec((_BLK, _HID), lambda i: (i, 0)),
            pl.BlockSpec((_HID, _HID), lambda i: (0, 0)),
            pl.BlockSpec((1, _HID), lambda i: (0, 0)),
            pl.BlockSpec((_HID, _HID), lambda i: (0, 0)),
            pl.BlockSpec((_BLK, 1), lambda i: (i, 0)),
        ],
        out_specs=pl.BlockSpec((_BLK, _HID), lambda i: (i, 0)),
        out_shape=jax.ShapeDtypeStruct((n, _HID), jnp.float32),
    )(mean, h, wrel, brel.reshape(1, _HID), wroot, maskf)


def _score_kernel(h_ref, p_ref, o_ref):
    o_ref[...] = jnp.sum(h_ref[...] * p_ref[...], axis=1, keepdims=True)


def _score(h, p):
    n = h.shape[0]
    return pl.pallas_call(
        _score_kernel,
        grid=(n // _BLK,),
        in_specs=[
            pl.BlockSpec((_BLK, _HID), lambda i: (i, 0)),
            pl.BlockSpec((1, _HID), lambda i: (0, 0)),
        ],
        out_specs=pl.BlockSpec((_BLK, 1), lambda i: (i, 0)),
        out_shape=jax.ShapeDtypeStruct((n, 1), jnp.float32),
    )(h, p.reshape(1, _HID))


def _head_kernel(xc_ref, w1_ref, b1_ref, w2_ref, b2_ref, o_ref):
    z = jnp.maximum(
        jnp.dot(xc_ref[...], w1_ref[...].T, preferred_element_type=jnp.float32)
        + b1_ref[...],
        0.0,
    )
    out = jnp.dot(z, w2_ref[...].T, preferred_element_type=jnp.float32) + b2_ref[...]
    m = jnp.max(out, axis=1, keepdims=True)
    lse = m + jnp.log(jnp.sum(jnp.exp(out - m), axis=1, keepdims=True))
    o_ref[...] = out - lse


def _head(xc, w1, b1, w2, b2):
    return pl.pallas_call(
        _head_kernel,
        out_shape=jax.ShapeDtypeStruct((xc.shape[0], w2.shape[0]), jnp.float32),
    )(xc, w1, b1.reshape(1, -1), w2, b2.reshape(1, -1))


def _agg_mean(h, src, dst, nmf, n):
    agg = jax.ops.segment_sum(h[src], dst, num_segments=n)
    deg = jax.ops.segment_sum(nmf[src], dst, num_segments=n)
    return agg / jnp.clip(deg, 1.0)[:, None]


def _gmp(h, batch, maskf):
    s = jax.ops.segment_sum(h, batch, num_segments=_NG)
    c = jax.ops.segment_sum(maskf[:, 0], batch, num_segments=_NG)
    return s / jnp.clip(c, 1.0)[:, None]


def _topk(h, p, batch, node_mask, starts):
    n = h.shape[0]
    score = _score(h, p)[:, 0] / (jnp.linalg.norm(p) + 1e-16)
    score_m = jnp.where(node_mask, score, -jnp.inf)
    o1 = jnp.argsort(-score_m)
    order = o1[jnp.argsort(batch[o1])]
    pos = jnp.zeros(n, dtype=jnp.int32).at[order].set(jnp.arange(n, dtype=jnp.int32))
    rank = pos - starts[batch]
    active = jax.ops.segment_sum(node_mask.astype(jnp.float32), batch, num_segments=_NG)
    k = jnp.ceil(_RATIO * active).astype(jnp.int32)
    keep = (rank < k[batch]) & node_mask
    x_new = h * jnp.tanh(score)[:, None] * keep[:, None].astype(h.dtype)
    return x_new, keep


def kernel(x, params, edge_index, batch):
    src, dst = edge_index[0], edge_index[1]
    n = x.shape[0]
    cnt = jnp.bincount(batch, length=_NG)
    starts = (jnp.cumsum(cnt) - cnt).astype(jnp.int32)
    node_mask = jnp.ones(n, dtype=bool)
    maskf = node_mask.astype(jnp.float32)[:, None]

    wrel1 = jnp.zeros((_HID, _HID), jnp.float32).at[:, 0].set(params["Wrel1"][:, 0])
    wroot1 = jnp.zeros((_HID, _HID), jnp.float32).at[:, 0].set(params["Wroot1"][:, 0])
    mean0 = jnp.pad(_agg_mean(x, src, dst, maskf[:, 0], n), ((0, 0), (0, _HID - 1)))
    xp = jnp.pad(x, ((0, 0), (0, _HID - 1)))
    h = _conv_dense(mean0, xp, wrel1, params["brel1"], wroot1, maskf)

    xs = [_gmp(h, batch, maskf)]
    for i in range(_NL - 1):
        mean = _agg_mean(h, src, dst, maskf[:, 0], n)
        h = _conv_dense(mean, h, params["Wrel"][i], params["brel"][i], params["Wroot"][i], maskf)
        xs.append(_gmp(h, batch, maskf))
        if i % 2 == 0 and i < _NL - 2:
            h, node_mask = _topk(h, params["pool_p"][i // 2], batch, node_mask, starts)
            maskf = node_mask.astype(jnp.float32)[:, None]

    xc = jnp.concatenate(xs, axis=1)
    return _head(xc, params["W_lin1"], params["b_lin1"], params["W_lin2"], params["b_lin2"])
